# baseline (speedup 1.0000x reference)
import functools

import jax
import jax.numpy as jnp
import numpy as np
from jax import lax
from jax.experimental import pallas as pl
from jax.experimental.pallas import tpu as pltpu
from jax.experimental.pallas import tpu_sc as plsc

VOCAB = 1000000
D = 64
B = 4096
L = 200
SCALE = 8.0

NC = 2
NS = 16
NW = NC * NS
BBLK = B // NW
TOTAL = B * L
XROWS = TOTAL // 128
XR_PER_W = XROWS // NW


def _embed_body(x_hbm, tab_hbm, out_hbm, xb, pbuf, gbuf, obuf, gsem, osem):
    c = lax.axis_index("c")
    s = lax.axis_index("s")
    w = s * NC + c
    lanes = lax.iota(jnp.int32, 16)

    pltpu.sync_copy(x_hbm.at[pl.ds(w * XR_PER_W, XR_PER_W)], xb)

    def l_body(l, carry):
        for sg in range(8):
            t = (sg * 16 + lanes) * L + l
            idx = plsc.load_gather(xb, [t >> 7, t & 127])
            pbuf[pl.ds(sg * 16, 16)] = idx
        pltpu.async_copy(tab_hbm.at[pbuf], gbuf, gsem).wait()

        def sg_body(sg, carry2):
            rows = sg * 16 + lanes
            for d in range(D):
                col = jnp.full((16,), d, jnp.int32)
                v = plsc.load_gather(gbuf, [rows, col])
                obuf[d >> 3, d & 7, pl.ds(sg * 16, 16)] = v * SCALE
            return carry2

        lax.fori_loop(0, 8, sg_body, 0)
        pltpu.async_copy(obuf, out_hbm.at[l, :, w], osem).wait()
        return carry

    lax.fori_loop(0, L, l_body, 0)


@jax.jit
def _embed(x2d, tpad):
    mesh = plsc.VectorSubcoreMesh(
        core_axis_name="c", subcore_axis_name="s", num_cores=NC, num_subcores=NS
    )
    return pl.kernel(
        _embed_body,
        out_type=jax.ShapeDtypeStruct((L, 8, NW, 8, BBLK), jnp.float32),
        mesh=mesh,
        scratch_types=[
            pltpu.VMEM((XR_PER_W, 128), jnp.int32),
            pltpu.VMEM((128,), jnp.int32),
            pltpu.VMEM((128, 128), jnp.float32),
            pltpu.VMEM((8, 8, BBLK), jnp.float32),
            pltpu.SemaphoreType.DMA,
            pltpu.SemaphoreType.DMA,
        ],
        compiler_params=pltpu.CompilerParams(needs_layout_passes=False),
    )(x2d, tpad)


def kernel(x, table):
    x2d = x.astype(jnp.int32).reshape(XROWS, 128)
    tpad = jnp.pad(table, ((0, 0), (0, 128 - D)))
    out5d = _embed(x2d, tpad)
    return out5d.transpose(2, 4, 0, 1, 3).reshape(B, L, D)


# speedup vs baseline: 1.3425x; 1.3425x over previous
import functools

import jax
import jax.numpy as jnp
import numpy as np
from jax import lax
from jax.experimental import pallas as pl
from jax.experimental.pallas import tpu as pltpu
from jax.experimental.pallas import tpu_sc as plsc

VOCAB = 1000000
D = 64
B = 4096
L = 200
SCALE = 8.0

NC = 2
NS = 16
NW = NC * NS
BBLK = B // NW
TOTAL = B * L
XROWS = TOTAL // 128
XR_PER_W = XROWS // NW
OPAD = 133


def _embed_body(
    x_hbm, tab_hbm, out_hbm,
    xb, pb0, pb1, gb0, gb1, ob0, ob1,
    gs0, gs1, os0, os1,
):
    c = lax.axis_index("c")
    s = lax.axis_index("s")
    w = s * NC + c
    lanes = lax.iota(jnp.int32, 16)
    pbufs, gbufs, obufs = (pb0, pb1), (gb0, gb1), (ob0, ob1)
    gsems, osems = (gs0, gs1), (os0, os1)

    d_hi = [(16 * k + lanes) >> 3 for k in range(4)]
    d_lo = [(16 * k + lanes) & 7 for k in range(4)]

    pltpu.sync_copy(x_hbm.at[pl.ds(w * XR_PER_W, XR_PER_W)], xb)

    def stage(l, pb):
        for sg in range(8):
            t = (sg * 16 + lanes) * L + l
            pb[pl.ds(sg * 16, 16)] = plsc.load_gather(xb, [t >> 7, t & 127])

    stage(0, pb0)
    pltpu.async_copy(tab_hbm.at[pb0], gb0, gs0)

    def outer(i, carry):
        for par in range(2):
            l = i * 2 + par
            pb, gb, ob = pbufs[par], gbufs[par], obufs[par]
            pltpu.make_async_copy(tab_hbm.at[pb], gb, gsems[par]).wait()

            @pl.when(l < L - 1)
            def _():
                stage(l + 1, pbufs[1 - par])
                pltpu.async_copy(
                    tab_hbm.at[pbufs[1 - par]], gbufs[1 - par], gsems[1 - par]
                )

            @pl.when(l >= 2)
            def _():
                pltpu.make_async_copy(
                    ob.at[:, :, pl.ds(0, BBLK)], out_hbm.at[l, :, w], osems[par]
                ).wait()

            def jbody(j, carry2):
                col = jnp.full((16,), j, jnp.int32)
                for k in range(4):
                    v = gb[j, pl.ds(k * 16, 16)]
                    plsc.store_scatter(ob, [d_hi[k], d_lo[k], col], v * SCALE)
                return carry2

            lax.fori_loop(0, BBLK, jbody, 0, unroll=4)
            pltpu.async_copy(
                ob.at[:, :, pl.ds(0, BBLK)], out_hbm.at[l, :, w], osems[par]
            )
        return carry

    lax.fori_loop(0, L // 2, outer, 0)
    pltpu.make_async_copy(
        ob0.at[:, :, pl.ds(0, BBLK)], out_hbm.at[L - 2, :, w], os0
    ).wait()
    pltpu.make_async_copy(
        ob1.at[:, :, pl.ds(0, BBLK)], out_hbm.at[L - 1, :, w], os1
    ).wait()


@jax.jit
def _embed(x2d, tpad):
    mesh = plsc.VectorSubcoreMesh(
        core_axis_name="c", subcore_axis_name="s", num_cores=NC, num_subcores=NS
    )
    return pl.kernel(
        _embed_body,
        out_type=jax.ShapeDtypeStruct((L, 8, NW, 8, BBLK), jnp.float32),
        mesh=mesh,
        scratch_types=[
            pltpu.VMEM((XR_PER_W, 128), jnp.int32),
            pltpu.VMEM((128,), jnp.int32),
            pltpu.VMEM((128,), jnp.int32),
            pltpu.VMEM((128, 128), jnp.float32),
            pltpu.VMEM((128, 128), jnp.float32),
            pltpu.VMEM((8, 8, OPAD), jnp.float32),
            pltpu.VMEM((8, 8, OPAD), jnp.float32),
            pltpu.SemaphoreType.DMA,
            pltpu.SemaphoreType.DMA,
            pltpu.SemaphoreType.DMA,
            pltpu.SemaphoreType.DMA,
        ],
        compiler_params=pltpu.CompilerParams(needs_layout_passes=False),
    )(x2d, tpad)


def kernel(x, table):
    x2d = x.astype(jnp.int32).reshape(XROWS, 128)
    tpad = jnp.pad(table, ((0, 0), (0, 128 - D)))
    out5d = _embed(x2d, tpad)
    return out5d.transpose(2, 4, 0, 1, 3).reshape(B, L, D)


# speedup vs baseline: 1.7827x; 1.3279x over previous
import functools

import jax
import jax.numpy as jnp
import numpy as np
from jax import lax
from jax.experimental import pallas as pl
from jax.experimental.pallas import tpu as pltpu
from jax.experimental.pallas import tpu_sc as plsc

VOCAB = 1000000
D = 64
B = 4096
L = 200
SCALE = 8.0

NC = 2
NS = 16
NW = NC * NS
BBLK = B // NW
TOTAL = B * L
XROWS = TOTAL // 128
XR_PER_W = XROWS // NW
OPAD = 133


def _embed_body(
    x_hbm, tab_hbm, out_hbm,
    xb, pb0, pb1, gb0, gb1, ob0, ob1,
    gs0, gs1, os0, os1,
):
    c = lax.axis_index("c")
    s = lax.axis_index("s")
    w = s * NC + c
    lanes = lax.iota(jnp.int32, 16)
    pbufs, gbufs, obufs = (pb0, pb1), (gb0, gb1), (ob0, ob1)
    gsems, osems = (gs0, gs1), (os0, os1)

    d_hi = [(16 * k + lanes) >> 3 for k in range(4)]
    d_lo = [(16 * k + lanes) & 7 for k in range(4)]

    pltpu.sync_copy(x_hbm.at[pl.ds(w * XR_PER_W, XR_PER_W)], xb)

    def stage(l, pb):
        for sg in range(8):
            t = (sg * 16 + lanes) * L + l
            pb[pl.ds(sg * 16, 16)] = plsc.load_gather(xb, [t >> 7, t & 127])

    stage(0, pb0)
    pltpu.async_copy(tab_hbm.at[pb0], gb0, gs0)

    def outer(i, carry):
        for par in range(2):
            l = i * 2 + par
            pb, gb, ob = pbufs[par], gbufs[par], obufs[par]
            pltpu.make_async_copy(tab_hbm.at[pb], gb, gsems[par]).wait()

            @pl.when(l < L - 1)
            def _():
                stage(l + 1, pbufs[1 - par])
                pltpu.async_copy(
                    tab_hbm.at[pbufs[1 - par]], gbufs[1 - par], gsems[1 - par]
                )

            @pl.when(l >= 2)
            def _():
                pltpu.make_async_copy(
                    ob.at[:, :, pl.ds(0, BBLK)], out_hbm.at[l, :, w], osems[par]
                ).wait()

            @plsc.parallel_loop(0, BBLK, unroll=4)
            def _(j):
                col = jnp.full((16,), j, jnp.int32)
                for k in range(4):
                    v = gb[j, pl.ds(k * 16, 16)]
                    plsc.store_scatter(ob, [d_hi[k], d_lo[k], col], v * SCALE)
            pltpu.async_copy(
                ob.at[:, :, pl.ds(0, BBLK)], out_hbm.at[l, :, w], osems[par]
            )
        return carry

    lax.fori_loop(0, L // 2, outer, 0)
    pltpu.make_async_copy(
        ob0.at[:, :, pl.ds(0, BBLK)], out_hbm.at[L - 2, :, w], os0
    ).wait()
    pltpu.make_async_copy(
        ob1.at[:, :, pl.ds(0, BBLK)], out_hbm.at[L - 1, :, w], os1
    ).wait()


@jax.jit
def _embed(x2d, tpad):
    mesh = plsc.VectorSubcoreMesh(
        core_axis_name="c", subcore_axis_name="s", num_cores=NC, num_subcores=NS
    )
    return pl.kernel(
        _embed_body,
        out_type=jax.ShapeDtypeStruct((L, 8, NW, 8, BBLK), jnp.float32),
        mesh=mesh,
        scratch_types=[
            pltpu.VMEM((XR_PER_W, 128), jnp.int32),
            pltpu.VMEM((128,), jnp.int32),
            pltpu.VMEM((128,), jnp.int32),
            pltpu.VMEM((128, 128), jnp.float32),
            pltpu.VMEM((128, 128), jnp.float32),
            pltpu.VMEM((8, 8, OPAD), jnp.float32),
            pltpu.VMEM((8, 8, OPAD), jnp.float32),
            pltpu.SemaphoreType.DMA,
            pltpu.SemaphoreType.DMA,
            pltpu.SemaphoreType.DMA,
            pltpu.SemaphoreType.DMA,
        ],
        compiler_params=pltpu.CompilerParams(needs_layout_passes=False),
    )(x2d, tpad)


def kernel(x, table):
    x2d = x.astype(jnp.int32).reshape(XROWS, 128)
    tpad = jnp.pad(table, ((0, 0), (0, 128 - D)))
    out5d = _embed(x2d, tpad)
    return out5d.transpose(2, 4, 0, 1, 3).reshape(B, L, D)
